# Initial kernel scaffold; baseline (speedup 1.0000x reference)
import jax
import jax.numpy as jnp
from jax.experimental import pallas as pl


def kernel(x, edge_index, edge_weight, W_l, b_l):
    raise NotImplementedError("write your pallas kernel here")


# speedup vs baseline: 5.8410x; 5.8410x over previous
import functools

import jax
import jax.numpy as jnp
from jax import lax
from jax.experimental import pallas as pl
from jax.experimental.pallas import tpu as pltpu
from jax.experimental.pallas import tpu_sc as plsc

NC = 2
NS = 16
LANES = 16
CHUNK = 80


def _sc_aggregate(x, src, dst, w, n_nodes, d):
    e = src.shape[0]
    nw = NC * NS
    e_tile = e // nw
    nchunk = e_tile // CHUNK
    dblk = d // LANES
    rpt = (n_nodes // NS) // 8 * 8
    rpt_last = n_nodes - (NS - 1) * rpt

    zblk = jnp.zeros((rpt_last, d), dtype=jnp.float32)

    mesh = plsc.VectorSubcoreMesh(core_axis_name="c", subcore_axis_name="s")

    @functools.partial(
        pl.kernel,
        mesh=mesh,
        out_type=(
            jax.ShapeDtypeStruct((n_nodes, d), jnp.float32),
            jax.ShapeDtypeStruct((n_nodes, d), jnp.float32),
        ),
        scratch_types=[
            pltpu.VMEM((e_tile,), jnp.int32),
            pltpu.VMEM((e_tile,), jnp.float32),
            pltpu.VMEM((CHUNK,), jnp.int32),
            pltpu.VMEM((CHUNK, d), jnp.float32),
            pltpu.VMEM_SHARED((n_nodes, d), jnp.float32),
            pltpu.SemaphoreType.DMA,
        ],
    )
    def agg_kernel(x_hbm, src_hbm, dst_hbm, w_hbm, z_hbm, p0_hbm, p1_hbm,
                   src_v, w_v, dst_v, rows_v, acc, sem):
        cid = lax.axis_index("c")
        sid = lax.axis_index("s")
        wid = cid * NS + sid
        ebase = wid * e_tile
        rbase = sid * rpt

        @pl.when(sid == NS - 1)
        def _():
            pltpu.sync_copy(z_hbm, acc.at[pl.ds(rbase, rpt_last)])

        @pl.when(sid != NS - 1)
        def _():
            pltpu.sync_copy(z_hbm.at[pl.ds(0, rpt)], acc.at[pl.ds(rbase, rpt)])
        pltpu.sync_copy(src_hbm.at[pl.ds(ebase, e_tile)], src_v)
        pltpu.sync_copy(w_hbm.at[pl.ds(ebase, e_tile)], w_v)
        plsc.subcore_barrier()

        def chunk_body(c, carry):
            eoff = c * CHUNK
            pltpu.sync_copy(dst_hbm.at[pl.ds(ebase + eoff, CHUNK)], dst_v)
            pltpu.async_copy(
                x_hbm.at[src_v.at[pl.ds(eoff, CHUNK)]], rows_v, sem
            ).wait()

            def scale_group(g, carry2):
                w16 = w_v[pl.ds(eoff + g * LANES, LANES)]
                for t in range(LANES):
                    wt = w16[t]
                    i = g * LANES + t
                    for j in range(dblk):
                        sl = pl.ds(j * LANES, LANES)
                        rows_v[i, sl] = rows_v[i, sl] * wt
                return carry2

            lax.fori_loop(0, CHUNK // LANES, scale_group, 0, unroll=False)
            pltpu.sync_copy(rows_v, acc.at[dst_v], add=True)
            return carry

        lax.fori_loop(0, nchunk, chunk_body, 0, unroll=False)
        plsc.subcore_barrier()

        def copy_out(dest):
            @pl.when(sid == NS - 1)
            def _():
                pltpu.sync_copy(acc.at[pl.ds(rbase, rpt_last)],
                                dest.at[pl.ds(rbase, rpt_last)])

            @pl.when(sid != NS - 1)
            def _():
                pltpu.sync_copy(acc.at[pl.ds(rbase, rpt)],
                                dest.at[pl.ds(rbase, rpt)])

        @pl.when(cid == 0)
        def _():
            copy_out(p0_hbm)

        @pl.when(cid != 0)
        def _():
            copy_out(p1_hbm)

    return agg_kernel(x, src, dst, w, zblk)


def _tc_linear(p0, p1, W_l, b_l):
    n, d_in = p0.shape
    d_out = W_l.shape[1]
    blk = 2000

    def mm_body(p0_ref, p1_ref, w_ref, b_ref, o_ref):
        s = p0_ref[...] + p1_ref[...]
        o_ref[...] = (
            jnp.dot(s, w_ref[...], preferred_element_type=jnp.float32)
            + b_ref[...]
        )

    return pl.pallas_call(
        mm_body,
        grid=(n // blk,),
        in_specs=[
            pl.BlockSpec((blk, d_in), lambda i: (i, 0)),
            pl.BlockSpec((blk, d_in), lambda i: (i, 0)),
            pl.BlockSpec((d_in, d_out), lambda i: (0, 0)),
            pl.BlockSpec((1, d_out), lambda i: (0, 0)),
        ],
        out_specs=pl.BlockSpec((blk, d_out), lambda i: (i, 0)),
        out_shape=jax.ShapeDtypeStruct((n, d_out), jnp.float32),
    )(p0, p1, W_l, b_l.reshape(1, d_out))


def kernel(x, edge_index, edge_weight, W_l, b_l):
    n, d = x.shape
    dst = edge_index[0]
    src = edge_index[1]
    p0, p1 = _sc_aggregate(x, src, dst, edge_weight, n, d)
    return _tc_linear(p0, p1, W_l, b_l)


# speedup vs baseline: 11.1128x; 1.9025x over previous
import functools

import jax
import jax.numpy as jnp
from jax import lax
from jax.experimental import pallas as pl
from jax.experimental.pallas import tpu as pltpu
from jax.experimental.pallas import tpu_sc as plsc

NC = 2
NS = 16
LANES = 16
CHUNK = 80


def _sc_aggregate(x, src, dst, w, n_nodes, d):
    e = src.shape[0]
    nw = NC * NS
    e_tile = e // nw
    nchunk = e_tile // CHUNK
    dblk = d // LANES
    rpt = (n_nodes // NS) // 8 * 8
    rpt_last = n_nodes - (NS - 1) * rpt

    zblk = jnp.zeros((rpt_last, d), dtype=jnp.float32)

    mesh = plsc.VectorSubcoreMesh(core_axis_name="c", subcore_axis_name="s")

    @functools.partial(
        pl.kernel,
        mesh=mesh,
        out_type=(
            jax.ShapeDtypeStruct((n_nodes, d), jnp.float32),
            jax.ShapeDtypeStruct((n_nodes, d), jnp.float32),
        ),
        scratch_types=[
            pltpu.VMEM((e_tile,), jnp.int32),
            pltpu.VMEM((e_tile,), jnp.float32),
            pltpu.VMEM((CHUNK,), jnp.int32),
            pltpu.VMEM((CHUNK,), jnp.int32),
            pltpu.VMEM((CHUNK, d), jnp.float32),
            pltpu.VMEM((CHUNK, d), jnp.float32),
            pltpu.VMEM_SHARED((n_nodes, d), jnp.float32),
            pltpu.SemaphoreType.DMA,
            pltpu.SemaphoreType.DMA,
            pltpu.SemaphoreType.DMA,
            pltpu.SemaphoreType.DMA,
            pltpu.SemaphoreType.DMA,
            pltpu.SemaphoreType.DMA,
        ],
    )
    def agg_kernel(x_hbm, src_hbm, dst_hbm, w_hbm, z_hbm, p0_hbm, p1_hbm,
                   src_v, w_v, dst0_v, dst1_v, rows0_v, rows1_v, acc,
                   gsem0, gsem1, dsem0, dsem1, ssem0, ssem1):
        cid = lax.axis_index("c")
        sid = lax.axis_index("s")
        wid = cid * NS + sid
        ebase = wid * e_tile
        rbase = sid * rpt

        @pl.when(sid == NS - 1)
        def _():
            pltpu.sync_copy(z_hbm, acc.at[pl.ds(rbase, rpt_last)])

        @pl.when(sid != NS - 1)
        def _():
            pltpu.sync_copy(z_hbm.at[pl.ds(0, rpt)], acc.at[pl.ds(rbase, rpt)])
        pltpu.sync_copy(src_hbm.at[pl.ds(ebase, e_tile)], src_v)
        pltpu.sync_copy(w_hbm.at[pl.ds(ebase, e_tile)], w_v)
        plsc.subcore_barrier()

        bufs = ((dst0_v, rows0_v, gsem0, dsem0, ssem0),
                (dst1_v, rows1_v, gsem1, dsem1, ssem1))

        def issue_fetch(c, buf):
            dst_b, rows_b, gsem_b, dsem_b, _ = buf
            eoff = c * CHUNK
            pltpu.async_copy(dst_hbm.at[pl.ds(ebase + eoff, CHUNK)],
                             dst_b, dsem_b)
            pltpu.async_copy(x_hbm.at[src_v.at[pl.ds(eoff, CHUNK)]],
                             rows_b, gsem_b)

        def do_iter(c, cur, nxt):
            dst_b, rows_b, gsem_b, dsem_b, ssem_b = cur
            _, nrows_b, _, _, nssem_b = nxt
            eoff = c * CHUNK

            @pl.when(c >= 1)
            def _():
                pltpu.make_async_copy(
                    nrows_b, acc.at[nxt[0]], nssem_b).wait()

            @pl.when(c + 1 < nchunk)
            def _():
                issue_fetch(c + 1, nxt)

            pltpu.make_async_copy(
                x_hbm.at[src_v.at[pl.ds(eoff, CHUNK)]], rows_b, gsem_b
            ).wait()

            def scale_group(g, carry2):
                w16 = w_v[pl.ds(eoff + g * LANES, LANES)]
                for t in range(LANES):
                    wt = w16[t]
                    i = g * LANES + t
                    for j in range(dblk):
                        sl = pl.ds(j * LANES, LANES)
                        rows_b[i, sl] = rows_b[i, sl] * wt
                return carry2

            lax.fori_loop(0, CHUNK // LANES, scale_group, 0, unroll=False)

            pltpu.make_async_copy(dst_hbm.at[pl.ds(ebase + eoff, CHUNK)],
                                  dst_b, dsem_b).wait()
            pltpu.async_copy(rows_b, acc.at[dst_b], ssem_b, add=True)

        def chunk_body(c, carry):
            @pl.when(c % 2 == 0)
            def _():
                do_iter(c, bufs[0], bufs[1])

            @pl.when(c % 2 == 1)
            def _():
                do_iter(c, bufs[1], bufs[0])

            return carry

        issue_fetch(0, bufs[0])
        lax.fori_loop(0, nchunk, chunk_body, 0, unroll=False)
        last = bufs[(nchunk - 1) % 2]
        pltpu.make_async_copy(last[1], acc.at[last[0]], last[4]).wait()
        plsc.subcore_barrier()

        def copy_out(dest):
            @pl.when(sid == NS - 1)
            def _():
                pltpu.sync_copy(acc.at[pl.ds(rbase, rpt_last)],
                                dest.at[pl.ds(rbase, rpt_last)])

            @pl.when(sid != NS - 1)
            def _():
                pltpu.sync_copy(acc.at[pl.ds(rbase, rpt)],
                                dest.at[pl.ds(rbase, rpt)])

        @pl.when(cid == 0)
        def _():
            copy_out(p0_hbm)

        @pl.when(cid != 0)
        def _():
            copy_out(p1_hbm)

    return agg_kernel(x, src, dst, w, zblk)


def _tc_linear(p0, p1, W_l, b_l):
    n, d_in = p0.shape
    d_out = W_l.shape[1]
    blk = 2000

    def mm_body(p0_ref, p1_ref, w_ref, b_ref, o_ref):
        s = p0_ref[...] + p1_ref[...]
        o_ref[...] = (
            jnp.dot(s, w_ref[...], preferred_element_type=jnp.float32)
            + b_ref[...]
        )

    return pl.pallas_call(
        mm_body,
        grid=(n // blk,),
        in_specs=[
            pl.BlockSpec((blk, d_in), lambda i: (i, 0)),
            pl.BlockSpec((blk, d_in), lambda i: (i, 0)),
            pl.BlockSpec((d_in, d_out), lambda i: (0, 0)),
            pl.BlockSpec((1, d_out), lambda i: (0, 0)),
        ],
        out_specs=pl.BlockSpec((blk, d_out), lambda i: (i, 0)),
        out_shape=jax.ShapeDtypeStruct((n, d_out), jnp.float32),
    )(p0, p1, W_l, b_l.reshape(1, d_out))


def kernel(x, edge_index, edge_weight, W_l, b_l):
    n, d = x.shape
    dst = edge_index[0]
    src = edge_index[1]
    p0, p1 = _sc_aggregate(x, src, dst, edge_weight, n, d)
    return _tc_linear(p0, p1, W_l, b_l)
